# baseline (speedup 1.0000x reference)
import jax
import jax.numpy as jnp
from jax.experimental import pallas as pl
from jax.experimental.pallas import tpu as pltpu

_B, _H, _S, _D = 16, 16, 2048, 128
_L = 16


def _update_body(pos_ref, kv_ref, vv_ref, kc_ref, vc_ref, ko_ref, vo_ref):
    ko_ref[...] = kc_ref[...]
    vo_ref[...] = vc_ref[...]
    sub = jax.lax.broadcasted_iota(jnp.int32, (8, _D), 0)
    for l in range(_L):
        p = pos_ref[l]
        base = pl.multiple_of((p // 8) * 8, 8)
        off = p - base
        for ref, vref in ((ko_ref, kv_ref), (vo_ref, vv_ref)):
            chunk = ref[0, pl.ds(base, 8), :]
            row = vref[0, pl.ds(l, 1), :]
            ref[0, pl.ds(base, 8), :] = jnp.where(sub == off, row, chunk)


def kernel(input_pos, k_val, v_val, k_cache, v_cache):
    BH = _B * _H
    kc = k_cache.reshape(BH, _S, _D)
    vc = v_cache.reshape(BH, _S, _D)
    kv = k_val.reshape(BH, _L, _D)
    vv = v_val.reshape(BH, _L, _D)
    pos = input_pos.astype(jnp.int32)

    ko, vo = pl.pallas_call(
        _update_body,
        grid=(BH,),
        in_specs=[
            pl.BlockSpec(memory_space=pltpu.SMEM),
            pl.BlockSpec((1, _L, _D), lambda i: (i, 0, 0)),
            pl.BlockSpec((1, _L, _D), lambda i: (i, 0, 0)),
            pl.BlockSpec((1, _S, _D), lambda i: (i, 0, 0)),
            pl.BlockSpec((1, _S, _D), lambda i: (i, 0, 0)),
        ],
        out_specs=[
            pl.BlockSpec((1, _S, _D), lambda i: (i, 0, 0)),
            pl.BlockSpec((1, _S, _D), lambda i: (i, 0, 0)),
        ],
        out_shape=[jax.ShapeDtypeStruct((BH, _S, _D), kc.dtype)] * 2,
    )(pos, kv, vv, kc, vc)
    return ko.reshape(_B, _H, _S, _D), vo.reshape(_B, _H, _S, _D)
